# baseline (speedup 1.0000x reference)
import functools

import jax
import jax.numpy as jnp
from jax import lax
from jax.experimental import pallas as pl
from jax.experimental.pallas import tpu as pltpu
from jax.experimental.pallas import tpu_sc as plsc

B, N, C, STRIDE, K = 4, 8192, 128, 4, 16
OUT = 128
M = N // STRIDE
R, L = 64, 128
DP = 256
MB = 256
MB2 = 256
G2 = (B * M) // MB2
S = B * M * K
EPS = 1e-5


def _fps_body(pp_ref, p2_ref):
    px = pp_ref[:, 0]
    py = pp_ref[:, 1]
    pz = pp_ref[:, 2]
    flat = (lax.broadcasted_iota(jnp.int32, (1, R, L), 1) * L
            + lax.broadcasted_iota(jnp.int32, (1, R, L), 2))

    def get_q(last):
        msk = (flat == last).astype(jnp.float32)
        qx = jnp.sum(px * msk, axis=(1, 2), keepdims=True)
        qy = jnp.sum(py * msk, axis=(1, 2), keepdims=True)
        qz = jnp.sum(pz * msk, axis=(1, 2), keepdims=True)
        return qx, qy, qz

    def body(i, carry):
        dists, last = carry
        qx, qy, qz = get_q(last)
        p2_ref[:, pl.ds(i - 1, 1), 0:1] = qx
        p2_ref[:, pl.ds(i - 1, 1), 1:2] = qy
        p2_ref[:, pl.ds(i - 1, 1), 2:3] = qz
        d = (px - qx) ** 2 + (py - qy) ** 2 + (pz - qz) ** 2
        dists = jnp.minimum(dists, d)
        mx = jnp.max(dists, axis=(1, 2), keepdims=True)
        nxt = jnp.min(jnp.where(dists == mx, flat, N),
                      axis=(1, 2), keepdims=True)
        return dists, nxt

    dists0 = jnp.full((B, R, L), 1e10, dtype=jnp.float32)
    last0 = jnp.zeros((B, 1, 1), dtype=jnp.int32)
    _, last = lax.fori_loop(1, M, body, (dists0, last0))
    qx, qy, qz = get_q(last)
    p2_ref[:, M - 1:M, 0:1] = qx
    p2_ref[:, M - 1:M, 1:2] = qy
    p2_ref[:, M - 1:M, 2:3] = qz


def _knn_body(pp8_ref, q8_ref, out_ref):
    b = pl.program_id(0)
    q = q8_ref[0]
    P = pp8_ref[0]
    qn = jnp.sum(q * q, axis=1, keepdims=True)
    pn = jnp.sum(P * P, axis=0, keepdims=True)
    dot = jnp.dot(q, P, preferred_element_type=jnp.float32)
    D = qn + pn - 2.0 * dot
    iota = lax.broadcasted_iota(jnp.int32, (1, N), 1)
    lastv = jnp.full((MB, 1), -jnp.inf, jnp.float32)
    lasti = jnp.full((MB, 1), -1, jnp.int32)
    for j in range(K):
        elig = (D > lastv) | ((D == lastv) & (iota > lasti))
        v = jnp.min(jnp.where(elig, D, jnp.inf), axis=1, keepdims=True)
        ix = jnp.min(jnp.where(elig & (D == v), iota, N), axis=1, keepdims=True)
        out_ref[0, :, j:j + 1] = ix + b * N
        lastv, lasti = v, ix


def _make_sc_gather(rows, dp):
    info = plsc.get_sparse_core_info()
    nw = info.num_cores * info.num_subcores
    per_w = rows // nw
    chunk = 128
    mesh = plsc.VectorSubcoreMesh(core_axis_name="c", subcore_axis_name="s")

    @functools.partial(
        pl.kernel,
        mesh=mesh,
        out_type=jax.ShapeDtypeStruct((rows, dp), jnp.float32),
        scratch_types=[
            pltpu.VMEM((chunk,), jnp.int32),
            pltpu.VMEM((chunk, dp), jnp.float32),
            pltpu.SemaphoreType.DMA,
        ],
    )
    def gather_k(table_hbm, idx_hbm, out_hbm, idx_v, rows_v, sem):
        wid = lax.axis_index("s") * info.num_cores + lax.axis_index("c")
        base = wid * per_w

        def body(ci, carry):
            off = base + ci * chunk
            pltpu.sync_copy(idx_hbm.at[pl.ds(off, chunk)], idx_v)
            pltpu.async_copy(table_hbm.at[idx_v], rows_v, sem).wait()
            pltpu.sync_copy(rows_v, out_hbm.at[pl.ds(off, chunk)])
            return carry

        lax.fori_loop(0, per_w // chunk, body, 0)

    return gather_k


def _mlp1_body(g_ref, q8_ref, w1p_ref, w1x_ref, h1_ref, st_ref):
    i = pl.program_id(0)
    g = g_ref[...]
    h = jnp.dot(g.reshape(MB2 * K, DP), w1p_ref[...],
                preferred_element_type=jnp.float32)
    corr = jnp.dot(q8_ref[...], w1x_ref[...],
                   preferred_element_type=jnp.float32)
    h = h.reshape(MB2, K, OUT) - corr[:, None, :]
    h1_ref[...] = h

    @pl.when(i == 0)
    def _():
        st_ref[...] = jnp.zeros((8, 128), jnp.float32)

    st_ref[0:1, :] = st_ref[0:1, :] + jnp.sum(h, axis=(0, 1))[None]
    st_ref[1:2, :] = st_ref[1:2, :] + jnp.sum(h * h, axis=(0, 1))[None]


def _mlp2_body(h1_ref, st1_ref, aux_ref, w2t_ref, hmx_ref, hmn_ref, st_ref):
    i = pl.program_id(0)
    inv = 1.0 / S
    mean1 = st1_ref[0:1, :] * inv
    var1 = st1_ref[1:2, :] * inv - mean1 * mean1
    s1 = aux_ref[0:1, :] * lax.rsqrt(var1 + EPS)
    t1 = aux_ref[1:2, :] - mean1 * s1
    h1 = h1_ref[...]
    a = jnp.maximum(h1 * s1[0] + t1[0], 0.0)
    h2 = jnp.dot(a.reshape(MB2 * K, OUT), w2t_ref[...],
                 preferred_element_type=jnp.float32).reshape(MB2, K, OUT)
    hmx_ref[...] = jnp.max(h2, axis=1)
    hmn_ref[...] = jnp.min(h2, axis=1)

    @pl.when(i == 0)
    def _():
        st_ref[...] = jnp.zeros((8, 128), jnp.float32)

    st_ref[0:1, :] = st_ref[0:1, :] + jnp.sum(h2, axis=(0, 1))[None]
    st_ref[1:2, :] = st_ref[1:2, :] + jnp.sum(h2 * h2, axis=(0, 1))[None]


def _final_body(hmx_ref, hmn_ref, st2_ref, aux_ref, y_ref):
    inv = 1.0 / S
    mean2 = st2_ref[0:1, :] * inv
    var2 = st2_ref[1:2, :] * inv - mean2 * mean2
    s2 = aux_ref[2:3, :] * lax.rsqrt(var2 + EPS)
    t2 = aux_ref[3:4, :] - mean2 * s2
    z = jnp.maximum(hmx_ref[...] * s2[0], hmn_ref[...] * s2[0])
    y_ref[...] = jnp.maximum(z + t2[0], 0.0)


def kernel(p1, x, W1, g1, b1, W2, g2, b2):
    f32 = jnp.float32
    pt = jnp.transpose(p1, (0, 2, 1))
    pp = pt.reshape(B, 3, R, L)
    pp8 = jnp.concatenate([pt, jnp.zeros((B, 5, N), f32)], axis=1)

    p2 = pl.pallas_call(
        _fps_body,
        out_shape=jax.ShapeDtypeStruct((B, M, 3), f32),
    )(pp)

    p2p8 = jnp.concatenate([p2, jnp.zeros((B, M, 5), f32)], axis=2)
    knn_flat = pl.pallas_call(
        _knn_body,
        grid=(B, M // MB),
        in_specs=[
            pl.BlockSpec((1, 8, N), lambda b, j: (b, 0, 0)),
            pl.BlockSpec((1, MB, 8), lambda b, j: (b, j, 0)),
        ],
        out_specs=pl.BlockSpec((1, MB, K), lambda b, j: (b, j, 0)),
        out_shape=jax.ShapeDtypeStruct((B, M, K), jnp.int32),
    )(pp8, p2p8)

    table = jnp.concatenate(
        [p1, jnp.transpose(x, (0, 2, 1)), jnp.zeros((B, N, DP - 3 - C), f32)],
        axis=2).reshape(B * N, DP)
    idx_flat = knn_flat.reshape(B * M * K)
    g = _make_sc_gather(B * M * K, DP)(table, idx_flat)
    g3 = g.reshape(B * M, K, DP)

    w1p = jnp.pad(W1, ((0, 0), (0, DP - 3 - C))).T
    w1x = jnp.pad(W1[:, :3], ((0, 0), (0, 5))).T
    q8f = p2p8.reshape(B * M, 8)
    h1, st1 = pl.pallas_call(
        _mlp1_body,
        grid=(G2,),
        in_specs=[
            pl.BlockSpec((MB2, K, DP), lambda i: (i, 0, 0)),
            pl.BlockSpec((MB2, 8), lambda i: (i, 0)),
            pl.BlockSpec((DP, OUT), lambda i: (0, 0)),
            pl.BlockSpec((8, OUT), lambda i: (0, 0)),
        ],
        out_specs=[
            pl.BlockSpec((MB2, K, OUT), lambda i: (i, 0, 0)),
            pl.BlockSpec((8, 128), lambda i: (0, 0)),
        ],
        out_shape=[
            jax.ShapeDtypeStruct((B * M, K, OUT), f32),
            jax.ShapeDtypeStruct((8, 128), f32),
        ],
    )(g3, q8f, w1p, w1x)

    aux = jnp.concatenate(
        [g1[None], b1[None], g2[None], b2[None],
         jnp.zeros((4, OUT), f32)], axis=0)
    hmx, hmn, st2 = pl.pallas_call(
        _mlp2_body,
        grid=(G2,),
        in_specs=[
            pl.BlockSpec((MB2, K, OUT), lambda i: (i, 0, 0)),
            pl.BlockSpec((8, 128), lambda i: (0, 0)),
            pl.BlockSpec((8, 128), lambda i: (0, 0)),
            pl.BlockSpec((OUT, OUT), lambda i: (0, 0)),
        ],
        out_specs=[
            pl.BlockSpec((MB2, OUT), lambda i: (i, 0)),
            pl.BlockSpec((MB2, OUT), lambda i: (i, 0)),
            pl.BlockSpec((8, 128), lambda i: (0, 0)),
        ],
        out_shape=[
            jax.ShapeDtypeStruct((B * M, OUT), f32),
            jax.ShapeDtypeStruct((B * M, OUT), f32),
            jax.ShapeDtypeStruct((8, 128), f32),
        ],
    )(h1, st1, aux, W2.T)

    y2 = pl.pallas_call(
        _final_body,
        grid=(G2,),
        in_specs=[
            pl.BlockSpec((MB2, OUT), lambda i: (i, 0)),
            pl.BlockSpec((MB2, OUT), lambda i: (i, 0)),
            pl.BlockSpec((8, 128), lambda i: (0, 0)),
            pl.BlockSpec((8, 128), lambda i: (0, 0)),
        ],
        out_specs=pl.BlockSpec((MB2, OUT), lambda i: (i, 0)),
        out_shape=jax.ShapeDtypeStruct((B * M, OUT), f32),
    )(hmx, hmn, st2, aux)

    y = jnp.transpose(y2.reshape(B, M, OUT), (0, 2, 1))
    return (p2, y)


# speedup vs baseline: 1.4741x; 1.4741x over previous
import functools

import jax
import jax.numpy as jnp
from jax import lax
from jax.experimental import pallas as pl
from jax.experimental.pallas import tpu as pltpu
from jax.experimental.pallas import tpu_sc as plsc

B, N, C, STRIDE, K = 4, 8192, 128, 4, 16
OUT = 128
M = N // STRIDE
R, L = 64, 128
DP = 256
MB = 128
MB2 = 256
G2 = (B * M) // MB2
S = B * M * K
EPS = 1e-5


def _fps_body(pp_ref, p2_ref):
    px = pp_ref[:, 0]
    py = pp_ref[:, 1]
    pz = pp_ref[:, 2]
    flat = (lax.broadcasted_iota(jnp.int32, (1, R, L), 1) * L
            + lax.broadcasted_iota(jnp.int32, (1, R, L), 2))

    def get_q(last):
        msk = (flat == last).astype(jnp.float32)
        qx = jnp.sum(px * msk, axis=(1, 2), keepdims=True)
        qy = jnp.sum(py * msk, axis=(1, 2), keepdims=True)
        qz = jnp.sum(pz * msk, axis=(1, 2), keepdims=True)
        return qx, qy, qz

    def body(i, carry):
        dists, last = carry
        qx, qy, qz = get_q(last)
        p2_ref[:, pl.ds(i - 1, 1), 0:1] = qx
        p2_ref[:, pl.ds(i - 1, 1), 1:2] = qy
        p2_ref[:, pl.ds(i - 1, 1), 2:3] = qz
        d = (px - qx) ** 2 + (py - qy) ** 2 + (pz - qz) ** 2
        dists = jnp.minimum(dists, d)
        mx = jnp.max(dists, axis=(1, 2), keepdims=True)
        nxt = jnp.min(jnp.where(dists == mx, flat, N),
                      axis=(1, 2), keepdims=True)
        return dists, nxt

    dists0 = jnp.full((B, R, L), 1e10, dtype=jnp.float32)
    last0 = jnp.zeros((B, 1, 1), dtype=jnp.int32)
    _, last = lax.fori_loop(1, M, body, (dists0, last0))
    qx, qy, qz = get_q(last)
    p2_ref[:, M - 1:M, 0:1] = qx
    p2_ref[:, M - 1:M, 1:2] = qy
    p2_ref[:, M - 1:M, 2:3] = qz


def _knn_body(pp8_ref, q8_ref, out_ref):
    b = pl.program_id(0)
    q = q8_ref[0]
    P = pp8_ref[0]
    qn = jnp.sum(q * q, axis=1, keepdims=True)
    pn = jnp.sum(P * P, axis=0, keepdims=True)
    dot = jnp.dot(q, P, preferred_element_type=jnp.float32)
    D = qn + pn - 2.0 * dot
    iota = lax.broadcasted_iota(jnp.int32, (1, N), 1)
    for j in range(K):
        mn = jnp.min(D, axis=1, keepdims=True)
        idx = jnp.min(jnp.where(D == mn, iota, N), axis=1, keepdims=True)
        out_ref[0, :, j:j + 1] = idx + b * N
        D = jnp.where(iota == idx, jnp.inf, D)


def _make_sc_gather(rows, dp):
    info = plsc.get_sparse_core_info()
    nw = info.num_cores * info.num_subcores
    per_w = rows // nw
    chunk = 128
    mesh = plsc.VectorSubcoreMesh(core_axis_name="c", subcore_axis_name="s")

    @functools.partial(
        pl.kernel,
        mesh=mesh,
        out_type=jax.ShapeDtypeStruct((rows, dp), jnp.float32),
        scratch_types=[
            pltpu.VMEM((chunk,), jnp.int32),
            pltpu.VMEM((chunk, dp), jnp.float32),
            pltpu.SemaphoreType.DMA,
        ],
    )
    def gather_k(table_hbm, idx_hbm, out_hbm, idx_v, rows_v, sem):
        wid = lax.axis_index("s") * info.num_cores + lax.axis_index("c")
        base = wid * per_w

        def body(ci, carry):
            off = base + ci * chunk
            pltpu.sync_copy(idx_hbm.at[pl.ds(off, chunk)], idx_v)
            pltpu.async_copy(table_hbm.at[idx_v], rows_v, sem).wait()
            pltpu.sync_copy(rows_v, out_hbm.at[pl.ds(off, chunk)])
            return carry

        lax.fori_loop(0, per_w // chunk, body, 0)

    return gather_k


def _mlp1_body(g_ref, q8_ref, w1p_ref, w1x_ref, h1_ref, st_ref):
    i = pl.program_id(0)
    g = g_ref[...]
    h = jnp.dot(g.reshape(MB2 * K, DP), w1p_ref[...],
                preferred_element_type=jnp.float32)
    corr = jnp.dot(q8_ref[...], w1x_ref[...],
                   preferred_element_type=jnp.float32)
    h = h.reshape(MB2, K, OUT) - corr[:, None, :]
    h1_ref[...] = h

    @pl.when(i == 0)
    def _():
        st_ref[...] = jnp.zeros((8, 128), jnp.float32)

    st_ref[0:1, :] = st_ref[0:1, :] + jnp.sum(h, axis=(0, 1))[None]
    st_ref[1:2, :] = st_ref[1:2, :] + jnp.sum(h * h, axis=(0, 1))[None]


def _mlp2_body(h1_ref, st1_ref, aux_ref, w2t_ref, hmx_ref, hmn_ref, st_ref):
    i = pl.program_id(0)
    inv = 1.0 / S
    mean1 = st1_ref[0:1, :] * inv
    var1 = st1_ref[1:2, :] * inv - mean1 * mean1
    s1 = aux_ref[0:1, :] * lax.rsqrt(var1 + EPS)
    t1 = aux_ref[1:2, :] - mean1 * s1
    h1 = h1_ref[...]
    a = jnp.maximum(h1 * s1[0] + t1[0], 0.0)
    h2 = jnp.dot(a.reshape(MB2 * K, OUT), w2t_ref[...],
                 preferred_element_type=jnp.float32).reshape(MB2, K, OUT)
    hmx_ref[...] = jnp.max(h2, axis=1)
    hmn_ref[...] = jnp.min(h2, axis=1)

    @pl.when(i == 0)
    def _():
        st_ref[...] = jnp.zeros((8, 128), jnp.float32)

    st_ref[0:1, :] = st_ref[0:1, :] + jnp.sum(h2, axis=(0, 1))[None]
    st_ref[1:2, :] = st_ref[1:2, :] + jnp.sum(h2 * h2, axis=(0, 1))[None]


def _final_body(hmx_ref, hmn_ref, st2_ref, aux_ref, y_ref):
    inv = 1.0 / S
    mean2 = st2_ref[0:1, :] * inv
    var2 = st2_ref[1:2, :] * inv - mean2 * mean2
    s2 = aux_ref[2:3, :] * lax.rsqrt(var2 + EPS)
    t2 = aux_ref[3:4, :] - mean2 * s2
    z = jnp.maximum(hmx_ref[...] * s2[0], hmn_ref[...] * s2[0])
    y_ref[...] = jnp.maximum(z + t2[0], 0.0)


def kernel(p1, x, W1, g1, b1, W2, g2, b2):
    f32 = jnp.float32
    pt = jnp.transpose(p1, (0, 2, 1))
    pp = pt.reshape(B, 3, R, L)
    pp8 = jnp.concatenate([pt, jnp.zeros((B, 5, N), f32)], axis=1)

    p2 = pl.pallas_call(
        _fps_body,
        out_shape=jax.ShapeDtypeStruct((B, M, 3), f32),
    )(pp)

    p2p8 = jnp.concatenate([p2, jnp.zeros((B, M, 5), f32)], axis=2)
    knn_flat = pl.pallas_call(
        _knn_body,
        grid=(B, M // MB),
        in_specs=[
            pl.BlockSpec((1, 8, N), lambda b, j: (b, 0, 0)),
            pl.BlockSpec((1, MB, 8), lambda b, j: (b, j, 0)),
        ],
        out_specs=pl.BlockSpec((1, MB, K), lambda b, j: (b, j, 0)),
        out_shape=jax.ShapeDtypeStruct((B, M, K), jnp.int32),
    )(pp8, p2p8)

    table = jnp.concatenate(
        [p1, jnp.transpose(x, (0, 2, 1)), jnp.zeros((B, N, DP - 3 - C), f32)],
        axis=2).reshape(B * N, DP)
    idx_flat = knn_flat.reshape(B * M * K)
    g = _make_sc_gather(B * M * K, DP)(table, idx_flat)
    g3 = g.reshape(B * M, K, DP)

    w1p = jnp.pad(W1, ((0, 0), (0, DP - 3 - C))).T
    w1x = jnp.pad(W1[:, :3], ((0, 0), (0, 5))).T
    q8f = p2p8.reshape(B * M, 8)
    h1, st1 = pl.pallas_call(
        _mlp1_body,
        grid=(G2,),
        in_specs=[
            pl.BlockSpec((MB2, K, DP), lambda i: (i, 0, 0)),
            pl.BlockSpec((MB2, 8), lambda i: (i, 0)),
            pl.BlockSpec((DP, OUT), lambda i: (0, 0)),
            pl.BlockSpec((8, OUT), lambda i: (0, 0)),
        ],
        out_specs=[
            pl.BlockSpec((MB2, K, OUT), lambda i: (i, 0, 0)),
            pl.BlockSpec((8, 128), lambda i: (0, 0)),
        ],
        out_shape=[
            jax.ShapeDtypeStruct((B * M, K, OUT), f32),
            jax.ShapeDtypeStruct((8, 128), f32),
        ],
    )(g3, q8f, w1p, w1x)

    aux = jnp.concatenate(
        [g1[None], b1[None], g2[None], b2[None],
         jnp.zeros((4, OUT), f32)], axis=0)
    hmx, hmn, st2 = pl.pallas_call(
        _mlp2_body,
        grid=(G2,),
        in_specs=[
            pl.BlockSpec((MB2, K, OUT), lambda i: (i, 0, 0)),
            pl.BlockSpec((8, 128), lambda i: (0, 0)),
            pl.BlockSpec((8, 128), lambda i: (0, 0)),
            pl.BlockSpec((OUT, OUT), lambda i: (0, 0)),
        ],
        out_specs=[
            pl.BlockSpec((MB2, OUT), lambda i: (i, 0)),
            pl.BlockSpec((MB2, OUT), lambda i: (i, 0)),
            pl.BlockSpec((8, 128), lambda i: (0, 0)),
        ],
        out_shape=[
            jax.ShapeDtypeStruct((B * M, OUT), f32),
            jax.ShapeDtypeStruct((B * M, OUT), f32),
            jax.ShapeDtypeStruct((8, 128), f32),
        ],
    )(h1, st1, aux, W2.T)

    y2 = pl.pallas_call(
        _final_body,
        grid=(G2,),
        in_specs=[
            pl.BlockSpec((MB2, OUT), lambda i: (i, 0)),
            pl.BlockSpec((MB2, OUT), lambda i: (i, 0)),
            pl.BlockSpec((8, 128), lambda i: (0, 0)),
            pl.BlockSpec((8, 128), lambda i: (0, 0)),
        ],
        out_specs=pl.BlockSpec((MB2, OUT), lambda i: (i, 0)),
        out_shape=jax.ShapeDtypeStruct((B * M, OUT), f32),
    )(hmx, hmn, st2, aux)

    y = jnp.transpose(y2.reshape(B, M, OUT), (0, 2, 1))
    return (p2, y)


# speedup vs baseline: 1.5717x; 1.0662x over previous
import functools

import jax
import jax.numpy as jnp
from jax import lax
from jax.experimental import pallas as pl
from jax.experimental.pallas import tpu as pltpu
from jax.experimental.pallas import tpu_sc as plsc

B, N, C, STRIDE, K = 4, 8192, 128, 4, 16
OUT = 128
M = N // STRIDE
R, L = 64, 128
DP = 256
MB = 128
MB2 = 256
G2 = (B * M) // MB2
S = B * M * K
EPS = 1e-5


def _fps_body(pp_ref, p2_ref):
    px = pp_ref[:, 0]
    py = pp_ref[:, 1]
    pz = pp_ref[:, 2]
    flat = (lax.broadcasted_iota(jnp.int32, (1, R, L), 1) * L
            + lax.broadcasted_iota(jnp.int32, (1, R, L), 2))

    def get_q(last):
        msk = (flat == last).astype(jnp.float32)
        qx = jnp.sum(px * msk, axis=(1, 2), keepdims=True)
        qy = jnp.sum(py * msk, axis=(1, 2), keepdims=True)
        qz = jnp.sum(pz * msk, axis=(1, 2), keepdims=True)
        return qx, qy, qz

    def body(i, carry):
        dists, last = carry
        qx, qy, qz = get_q(last)
        p2_ref[:, pl.ds(i - 1, 1), 0:1] = qx
        p2_ref[:, pl.ds(i - 1, 1), 1:2] = qy
        p2_ref[:, pl.ds(i - 1, 1), 2:3] = qz
        d = (px - qx) ** 2 + (py - qy) ** 2 + (pz - qz) ** 2
        dists = jnp.minimum(dists, d)
        mx = jnp.max(dists, axis=(1, 2), keepdims=True)
        nxt = jnp.min(jnp.where(dists == mx, flat, N),
                      axis=(1, 2), keepdims=True)
        return dists, nxt

    dists0 = jnp.full((B, R, L), 1e10, dtype=jnp.float32)
    last0 = jnp.zeros((B, 1, 1), dtype=jnp.int32)
    _, last = lax.fori_loop(1, M, body, (dists0, last0))
    qx, qy, qz = get_q(last)
    p2_ref[:, M - 1:M, 0:1] = qx
    p2_ref[:, M - 1:M, 1:2] = qy
    p2_ref[:, M - 1:M, 2:3] = qz


def _knn_body(pp8_ref, q8_ref, out_ref):
    b = pl.program_id(0)
    q = q8_ref[0]
    P = pp8_ref[0]
    qn = jnp.sum(q * q, axis=1, keepdims=True)
    pn = jnp.sum(P * P, axis=0, keepdims=True)
    dot = jnp.dot(q, P, preferred_element_type=jnp.float32)
    D = qn + pn - 2.0 * dot
    H = N // 4
    iota2 = lax.broadcasted_iota(jnp.int32, (1, H), 1)
    ch = [D[:, i * H:(i + 1) * H] for i in range(4)]
    rank = []
    for jj in range(4):
        r = jnp.zeros((MB, H), jnp.int32)
        for ii in range(4):
            if ii == jj:
                continue
            if ii < jj:
                r = r + (ch[ii] <= ch[jj]).astype(jnp.int32)
            else:
                r = r + (ch[ii] < ch[jj]).astype(jnp.int32)
        rank.append(r)
    vs, ns = [], []
    for rr, kk in enumerate((K, 8, 4, 1)):
        sv = jnp.full((MB, H), jnp.inf, jnp.float32)
        fv = jnp.zeros((MB, H), jnp.int32)
        for jj in range(4):
            m = rank[jj] == rr
            sv = jnp.where(m, ch[jj], sv)
            fv = jnp.where(m, iota2 + jj * H, fv)
        for j in range(kk):
            mn = jnp.min(sv, axis=1, keepdims=True)
            nx = jnp.min(jnp.where(sv == mn, fv, N), axis=1, keepdims=True)
            vs.append(mn)
            ns.append(nx)
            sv = jnp.where(fv == nx, jnp.inf, sv)
    Vc = jnp.concatenate(vs, axis=1)
    Nc = jnp.concatenate(ns, axis=1)
    for j in range(K):
        mn = jnp.min(Vc, axis=1, keepdims=True)
        nx = jnp.min(jnp.where(Vc == mn, Nc, N), axis=1, keepdims=True)
        out_ref[0, :, j:j + 1] = nx + b * N
        Vc = jnp.where((Vc == mn) & (Nc == nx), jnp.inf, Vc)


def _make_sc_gather(rows, dp):
    info = plsc.get_sparse_core_info()
    nw = info.num_cores * info.num_subcores
    per_w = rows // nw
    chunk = 128
    mesh = plsc.VectorSubcoreMesh(core_axis_name="c", subcore_axis_name="s")

    @functools.partial(
        pl.kernel,
        mesh=mesh,
        out_type=jax.ShapeDtypeStruct((rows, dp), jnp.float32),
        scratch_types=[
            pltpu.VMEM((chunk,), jnp.int32),
            pltpu.VMEM((chunk, dp), jnp.float32),
            pltpu.SemaphoreType.DMA,
        ],
    )
    def gather_k(table_hbm, idx_hbm, out_hbm, idx_v, rows_v, sem):
        wid = lax.axis_index("s") * info.num_cores + lax.axis_index("c")
        base = wid * per_w

        def body(ci, carry):
            off = base + ci * chunk
            pltpu.sync_copy(idx_hbm.at[pl.ds(off, chunk)], idx_v)
            pltpu.async_copy(table_hbm.at[idx_v], rows_v, sem).wait()
            pltpu.sync_copy(rows_v, out_hbm.at[pl.ds(off, chunk)])
            return carry

        lax.fori_loop(0, per_w // chunk, body, 0)

    return gather_k


def _mlp1_body(g_ref, q8_ref, w1p_ref, w1x_ref, h1_ref, st_ref):
    i = pl.program_id(0)
    g = g_ref[...]
    h = jnp.dot(g.reshape(MB2 * K, DP), w1p_ref[...],
                preferred_element_type=jnp.float32)
    corr = jnp.dot(q8_ref[...], w1x_ref[...],
                   preferred_element_type=jnp.float32)
    h = h.reshape(MB2, K, OUT) - corr[:, None, :]
    h1_ref[...] = h

    @pl.when(i == 0)
    def _():
        st_ref[...] = jnp.zeros((8, 128), jnp.float32)

    st_ref[0:1, :] = st_ref[0:1, :] + jnp.sum(h, axis=(0, 1))[None]
    st_ref[1:2, :] = st_ref[1:2, :] + jnp.sum(h * h, axis=(0, 1))[None]


def _mlp2_body(h1_ref, st1_ref, aux_ref, w2t_ref, hmx_ref, hmn_ref, st_ref):
    i = pl.program_id(0)
    inv = 1.0 / S
    mean1 = st1_ref[0:1, :] * inv
    var1 = st1_ref[1:2, :] * inv - mean1 * mean1
    s1 = aux_ref[0:1, :] * lax.rsqrt(var1 + EPS)
    t1 = aux_ref[1:2, :] - mean1 * s1
    h1 = h1_ref[...]
    a = jnp.maximum(h1 * s1[0] + t1[0], 0.0)
    h2 = jnp.dot(a.reshape(MB2 * K, OUT), w2t_ref[...],
                 preferred_element_type=jnp.float32).reshape(MB2, K, OUT)
    hmx_ref[...] = jnp.max(h2, axis=1)
    hmn_ref[...] = jnp.min(h2, axis=1)

    @pl.when(i == 0)
    def _():
        st_ref[...] = jnp.zeros((8, 128), jnp.float32)

    st_ref[0:1, :] = st_ref[0:1, :] + jnp.sum(h2, axis=(0, 1))[None]
    st_ref[1:2, :] = st_ref[1:2, :] + jnp.sum(h2 * h2, axis=(0, 1))[None]


def _final_body(hmx_ref, hmn_ref, st2_ref, aux_ref, y_ref):
    inv = 1.0 / S
    mean2 = st2_ref[0:1, :] * inv
    var2 = st2_ref[1:2, :] * inv - mean2 * mean2
    s2 = aux_ref[2:3, :] * lax.rsqrt(var2 + EPS)
    t2 = aux_ref[3:4, :] - mean2 * s2
    z = jnp.maximum(hmx_ref[...] * s2[0], hmn_ref[...] * s2[0])
    y_ref[...] = jnp.maximum(z + t2[0], 0.0)


def kernel(p1, x, W1, g1, b1, W2, g2, b2):
    f32 = jnp.float32
    pt = jnp.transpose(p1, (0, 2, 1))
    pp = pt.reshape(B, 3, R, L)
    pp8 = jnp.concatenate([pt, jnp.zeros((B, 5, N), f32)], axis=1)

    p2 = pl.pallas_call(
        _fps_body,
        out_shape=jax.ShapeDtypeStruct((B, M, 3), f32),
    )(pp)

    p2p8 = jnp.concatenate([p2, jnp.zeros((B, M, 5), f32)], axis=2)
    knn_flat = pl.pallas_call(
        _knn_body,
        grid=(B, M // MB),
        in_specs=[
            pl.BlockSpec((1, 8, N), lambda b, j: (b, 0, 0)),
            pl.BlockSpec((1, MB, 8), lambda b, j: (b, j, 0)),
        ],
        out_specs=pl.BlockSpec((1, MB, K), lambda b, j: (b, j, 0)),
        out_shape=jax.ShapeDtypeStruct((B, M, K), jnp.int32),
    )(pp8, p2p8)

    table = jnp.concatenate(
        [p1, jnp.transpose(x, (0, 2, 1)), jnp.zeros((B, N, DP - 3 - C), f32)],
        axis=2).reshape(B * N, DP)
    idx_flat = knn_flat.reshape(B * M * K)
    g = _make_sc_gather(B * M * K, DP)(table, idx_flat)
    g3 = g.reshape(B * M, K, DP)

    w1p = jnp.pad(W1, ((0, 0), (0, DP - 3 - C))).T
    w1x = jnp.pad(W1[:, :3], ((0, 0), (0, 5))).T
    q8f = p2p8.reshape(B * M, 8)
    h1, st1 = pl.pallas_call(
        _mlp1_body,
        grid=(G2,),
        in_specs=[
            pl.BlockSpec((MB2, K, DP), lambda i: (i, 0, 0)),
            pl.BlockSpec((MB2, 8), lambda i: (i, 0)),
            pl.BlockSpec((DP, OUT), lambda i: (0, 0)),
            pl.BlockSpec((8, OUT), lambda i: (0, 0)),
        ],
        out_specs=[
            pl.BlockSpec((MB2, K, OUT), lambda i: (i, 0, 0)),
            pl.BlockSpec((8, 128), lambda i: (0, 0)),
        ],
        out_shape=[
            jax.ShapeDtypeStruct((B * M, K, OUT), f32),
            jax.ShapeDtypeStruct((8, 128), f32),
        ],
    )(g3, q8f, w1p, w1x)

    aux = jnp.concatenate(
        [g1[None], b1[None], g2[None], b2[None],
         jnp.zeros((4, OUT), f32)], axis=0)
    hmx, hmn, st2 = pl.pallas_call(
        _mlp2_body,
        grid=(G2,),
        in_specs=[
            pl.BlockSpec((MB2, K, OUT), lambda i: (i, 0, 0)),
            pl.BlockSpec((8, 128), lambda i: (0, 0)),
            pl.BlockSpec((8, 128), lambda i: (0, 0)),
            pl.BlockSpec((OUT, OUT), lambda i: (0, 0)),
        ],
        out_specs=[
            pl.BlockSpec((MB2, OUT), lambda i: (i, 0)),
            pl.BlockSpec((MB2, OUT), lambda i: (i, 0)),
            pl.BlockSpec((8, 128), lambda i: (0, 0)),
        ],
        out_shape=[
            jax.ShapeDtypeStruct((B * M, OUT), f32),
            jax.ShapeDtypeStruct((B * M, OUT), f32),
            jax.ShapeDtypeStruct((8, 128), f32),
        ],
    )(h1, st1, aux, W2.T)

    y2 = pl.pallas_call(
        _final_body,
        grid=(G2,),
        in_specs=[
            pl.BlockSpec((MB2, OUT), lambda i: (i, 0)),
            pl.BlockSpec((MB2, OUT), lambda i: (i, 0)),
            pl.BlockSpec((8, 128), lambda i: (0, 0)),
            pl.BlockSpec((8, 128), lambda i: (0, 0)),
        ],
        out_specs=pl.BlockSpec((MB2, OUT), lambda i: (i, 0)),
        out_shape=jax.ShapeDtypeStruct((B * M, OUT), f32),
    )(hmx, hmn, st2, aux)

    y = jnp.transpose(y2.reshape(B, M, OUT), (0, 2, 1))
    return (p2, y)


# speedup vs baseline: 1.5937x; 1.0140x over previous
import functools

import jax
import jax.numpy as jnp
from jax import lax
from jax.experimental import pallas as pl
from jax.experimental.pallas import tpu as pltpu
from jax.experimental.pallas import tpu_sc as plsc

B, N, C, STRIDE, K = 4, 8192, 128, 4, 16
OUT = 128
M = N // STRIDE
R, L = 64, 128
DP = 256
MB = 128
MB2 = 512
G2 = (B * M) // MB2
S = B * M * K
EPS = 1e-5


def _fps_body(pp_ref, p2_ref):
    px = pp_ref[:, 0]
    py = pp_ref[:, 1]
    pz = pp_ref[:, 2]
    flat = (lax.broadcasted_iota(jnp.int32, (1, R, L), 1) * L
            + lax.broadcasted_iota(jnp.int32, (1, R, L), 2))

    def get_q(last):
        msk = (flat == last).astype(jnp.float32)
        qx = jnp.sum(px * msk, axis=(1, 2), keepdims=True)
        qy = jnp.sum(py * msk, axis=(1, 2), keepdims=True)
        qz = jnp.sum(pz * msk, axis=(1, 2), keepdims=True)
        return qx, qy, qz

    def body(i, carry):
        dists, last = carry
        qx, qy, qz = get_q(last)
        p2_ref[:, pl.ds(i - 1, 1), 0:1] = qx
        p2_ref[:, pl.ds(i - 1, 1), 1:2] = qy
        p2_ref[:, pl.ds(i - 1, 1), 2:3] = qz
        d = (px - qx) ** 2 + (py - qy) ** 2 + (pz - qz) ** 2
        dists = jnp.minimum(dists, d)
        mx = jnp.max(dists, axis=(1, 2), keepdims=True)
        nxt = jnp.min(jnp.where(dists == mx, flat, N),
                      axis=(1, 2), keepdims=True)
        return dists, nxt

    dists0 = jnp.full((B, R, L), 1e10, dtype=jnp.float32)
    last0 = jnp.zeros((B, 1, 1), dtype=jnp.int32)
    _, last = lax.fori_loop(1, M, body, (dists0, last0), unroll=2)
    qx, qy, qz = get_q(last)
    p2_ref[:, M - 1:M, 0:1] = qx
    p2_ref[:, M - 1:M, 1:2] = qy
    p2_ref[:, M - 1:M, 2:3] = qz


def _knn_body(pp8_ref, q8_ref, out_ref):
    b = pl.program_id(0)
    q = q8_ref[0]
    P = pp8_ref[0]
    qn = jnp.sum(q * q, axis=1, keepdims=True)
    pn = jnp.sum(P * P, axis=0, keepdims=True)
    dot = jnp.dot(q, P, preferred_element_type=jnp.float32)
    D = qn + pn - 2.0 * dot
    H = N // 4
    iota2 = lax.broadcasted_iota(jnp.int32, (1, H), 1)
    ch = [D[:, i * H:(i + 1) * H] for i in range(4)]
    rank = []
    for jj in range(4):
        r = jnp.zeros((MB, H), jnp.int32)
        for ii in range(4):
            if ii == jj:
                continue
            if ii < jj:
                r = r + (ch[ii] <= ch[jj]).astype(jnp.int32)
            else:
                r = r + (ch[ii] < ch[jj]).astype(jnp.int32)
        rank.append(r)
    vs, ns = [], []
    for rr, kk in enumerate((K, 8, 4, 1)):
        sv = jnp.full((MB, H), jnp.inf, jnp.float32)
        fv = jnp.zeros((MB, H), jnp.int32)
        for jj in range(4):
            m = rank[jj] == rr
            sv = jnp.where(m, ch[jj], sv)
            fv = jnp.where(m, iota2 + jj * H, fv)
        for j in range(kk):
            mn = jnp.min(sv, axis=1, keepdims=True)
            nx = jnp.min(jnp.where(sv == mn, fv, N), axis=1, keepdims=True)
            vs.append(mn)
            ns.append(nx)
            sv = jnp.where(fv == nx, jnp.inf, sv)
    Vc = jnp.concatenate(vs, axis=1)
    Nc = jnp.concatenate(ns, axis=1)
    for j in range(K):
        mn = jnp.min(Vc, axis=1, keepdims=True)
        nx = jnp.min(jnp.where(Vc == mn, Nc, N), axis=1, keepdims=True)
        out_ref[0, :, j:j + 1] = nx + b * N
        Vc = jnp.where((Vc == mn) & (Nc == nx), jnp.inf, Vc)


def _make_sc_gather(rows, dp):
    info = plsc.get_sparse_core_info()
    nw = info.num_cores * info.num_subcores
    per_w = rows // nw
    chunk = 128
    mesh = plsc.VectorSubcoreMesh(core_axis_name="c", subcore_axis_name="s")

    @functools.partial(
        pl.kernel,
        mesh=mesh,
        out_type=jax.ShapeDtypeStruct((rows, dp), jnp.float32),
        scratch_types=[
            pltpu.VMEM((chunk,), jnp.int32),
            pltpu.VMEM((chunk, dp), jnp.float32),
            pltpu.SemaphoreType.DMA,
        ],
    )
    def gather_k(table_hbm, idx_hbm, out_hbm, idx_v, rows_v, sem):
        wid = lax.axis_index("s") * info.num_cores + lax.axis_index("c")
        base = wid * per_w

        def body(ci, carry):
            off = base + ci * chunk
            pltpu.sync_copy(idx_hbm.at[pl.ds(off, chunk)], idx_v)
            pltpu.async_copy(table_hbm.at[idx_v], rows_v, sem).wait()
            pltpu.sync_copy(rows_v, out_hbm.at[pl.ds(off, chunk)])
            return carry

        lax.fori_loop(0, per_w // chunk, body, 0)

    return gather_k


def _mlp1_body(g_ref, q8_ref, w1p_ref, w1x_ref, h1_ref, st_ref):
    i = pl.program_id(0)
    g = g_ref[...]
    h = jnp.dot(g.reshape(MB2 * K, DP), w1p_ref[...],
                preferred_element_type=jnp.float32)
    corr = jnp.dot(q8_ref[...], w1x_ref[...],
                   preferred_element_type=jnp.float32)
    h = h.reshape(MB2, K, OUT) - corr[:, None, :]
    h1_ref[...] = h

    @pl.when(i == 0)
    def _():
        st_ref[...] = jnp.zeros((8, 128), jnp.float32)

    st_ref[0:1, :] = st_ref[0:1, :] + jnp.sum(h, axis=(0, 1))[None]
    st_ref[1:2, :] = st_ref[1:2, :] + jnp.sum(h * h, axis=(0, 1))[None]


def _mlp2_body(h1_ref, st1_ref, aux_ref, w2t_ref, hmx_ref, hmn_ref, st_ref):
    i = pl.program_id(0)
    inv = 1.0 / S
    mean1 = st1_ref[0:1, :] * inv
    var1 = st1_ref[1:2, :] * inv - mean1 * mean1
    s1 = aux_ref[0:1, :] * lax.rsqrt(var1 + EPS)
    t1 = aux_ref[1:2, :] - mean1 * s1
    h1 = h1_ref[...]
    a = jnp.maximum(h1 * s1[0] + t1[0], 0.0)
    h2 = jnp.dot(a.reshape(MB2 * K, OUT), w2t_ref[...],
                 preferred_element_type=jnp.float32).reshape(MB2, K, OUT)
    hmx_ref[...] = jnp.max(h2, axis=1)
    hmn_ref[...] = jnp.min(h2, axis=1)

    @pl.when(i == 0)
    def _():
        st_ref[...] = jnp.zeros((8, 128), jnp.float32)

    st_ref[0:1, :] = st_ref[0:1, :] + jnp.sum(h2, axis=(0, 1))[None]
    st_ref[1:2, :] = st_ref[1:2, :] + jnp.sum(h2 * h2, axis=(0, 1))[None]


def _final_body(hmx_ref, hmn_ref, st2_ref, aux_ref, y_ref):
    inv = 1.0 / S
    mean2 = st2_ref[0:1, :] * inv
    var2 = st2_ref[1:2, :] * inv - mean2 * mean2
    s2 = aux_ref[2:3, :] * lax.rsqrt(var2 + EPS)
    t2 = aux_ref[3:4, :] - mean2 * s2
    z = jnp.maximum(hmx_ref[...] * s2[0], hmn_ref[...] * s2[0])
    y_ref[...] = jnp.maximum(z + t2[0], 0.0)


def kernel(p1, x, W1, g1, b1, W2, g2, b2):
    f32 = jnp.float32
    pt = jnp.transpose(p1, (0, 2, 1))
    pp = pt.reshape(B, 3, R, L)
    pp8 = jnp.concatenate([pt, jnp.zeros((B, 5, N), f32)], axis=1)

    p2 = pl.pallas_call(
        _fps_body,
        out_shape=jax.ShapeDtypeStruct((B, M, 3), f32),
    )(pp)

    p2p8 = jnp.concatenate([p2, jnp.zeros((B, M, 5), f32)], axis=2)
    knn_flat = pl.pallas_call(
        _knn_body,
        grid=(B, M // MB),
        in_specs=[
            pl.BlockSpec((1, 8, N), lambda b, j: (b, 0, 0)),
            pl.BlockSpec((1, MB, 8), lambda b, j: (b, j, 0)),
        ],
        out_specs=pl.BlockSpec((1, MB, K), lambda b, j: (b, j, 0)),
        out_shape=jax.ShapeDtypeStruct((B, M, K), jnp.int32),
    )(pp8, p2p8)

    table = jnp.concatenate(
        [p1, jnp.transpose(x, (0, 2, 1)), jnp.zeros((B, N, DP - 3 - C), f32)],
        axis=2).reshape(B * N, DP)
    idx_flat = knn_flat.reshape(B * M * K)
    g = _make_sc_gather(B * M * K, DP)(table, idx_flat)
    g3 = g.reshape(B * M, K, DP)

    w1p = jnp.pad(W1, ((0, 0), (0, DP - 3 - C))).T
    w1x = jnp.pad(W1[:, :3], ((0, 0), (0, 5))).T
    q8f = p2p8.reshape(B * M, 8)
    h1, st1 = pl.pallas_call(
        _mlp1_body,
        grid=(G2,),
        in_specs=[
            pl.BlockSpec((MB2, K, DP), lambda i: (i, 0, 0)),
            pl.BlockSpec((MB2, 8), lambda i: (i, 0)),
            pl.BlockSpec((DP, OUT), lambda i: (0, 0)),
            pl.BlockSpec((8, OUT), lambda i: (0, 0)),
        ],
        out_specs=[
            pl.BlockSpec((MB2, K, OUT), lambda i: (i, 0, 0)),
            pl.BlockSpec((8, 128), lambda i: (0, 0)),
        ],
        out_shape=[
            jax.ShapeDtypeStruct((B * M, K, OUT), f32),
            jax.ShapeDtypeStruct((8, 128), f32),
        ],
    )(g3, q8f, w1p, w1x)

    aux = jnp.concatenate(
        [g1[None], b1[None], g2[None], b2[None],
         jnp.zeros((4, OUT), f32)], axis=0)
    hmx, hmn, st2 = pl.pallas_call(
        _mlp2_body,
        grid=(G2,),
        in_specs=[
            pl.BlockSpec((MB2, K, OUT), lambda i: (i, 0, 0)),
            pl.BlockSpec((8, 128), lambda i: (0, 0)),
            pl.BlockSpec((8, 128), lambda i: (0, 0)),
            pl.BlockSpec((OUT, OUT), lambda i: (0, 0)),
        ],
        out_specs=[
            pl.BlockSpec((MB2, OUT), lambda i: (i, 0)),
            pl.BlockSpec((MB2, OUT), lambda i: (i, 0)),
            pl.BlockSpec((8, 128), lambda i: (0, 0)),
        ],
        out_shape=[
            jax.ShapeDtypeStruct((B * M, OUT), f32),
            jax.ShapeDtypeStruct((B * M, OUT), f32),
            jax.ShapeDtypeStruct((8, 128), f32),
        ],
    )(h1, st1, aux, W2.T)

    y2 = pl.pallas_call(
        _final_body,
        grid=(G2,),
        in_specs=[
            pl.BlockSpec((MB2, OUT), lambda i: (i, 0)),
            pl.BlockSpec((MB2, OUT), lambda i: (i, 0)),
            pl.BlockSpec((8, 128), lambda i: (0, 0)),
            pl.BlockSpec((8, 128), lambda i: (0, 0)),
        ],
        out_specs=pl.BlockSpec((MB2, OUT), lambda i: (i, 0)),
        out_shape=jax.ShapeDtypeStruct((B * M, OUT), f32),
    )(hmx, hmn, st2, aux)

    y = jnp.transpose(y2.reshape(B, M, OUT), (0, 2, 1))
    return (p2, y)


# speedup vs baseline: 1.6943x; 1.0632x over previous
import functools

import jax
import jax.numpy as jnp
from jax import lax
from jax.experimental import pallas as pl
from jax.experimental.pallas import tpu as pltpu
from jax.experimental.pallas import tpu_sc as plsc

B, N, C, STRIDE, K = 4, 8192, 128, 4, 16
OUT = 128
M = N // STRIDE
R, L = 64, 128
DP = 256
MB = 256
MB2 = 512
G2 = (B * M) // MB2
S = B * M * K
EPS = 1e-5


def _fps_body(pp_ref, p2_ref):
    px = pp_ref[:, 0]
    py = pp_ref[:, 1]
    pz = pp_ref[:, 2]
    flat = (lax.broadcasted_iota(jnp.int32, (1, R, L), 1) * L
            + lax.broadcasted_iota(jnp.int32, (1, R, L), 2))

    def get_q(last):
        msk = (flat == last).astype(jnp.float32)
        qx = jnp.sum(px * msk, axis=(1, 2), keepdims=True)
        qy = jnp.sum(py * msk, axis=(1, 2), keepdims=True)
        qz = jnp.sum(pz * msk, axis=(1, 2), keepdims=True)
        return qx, qy, qz

    def body(i, carry):
        dists, last = carry
        qx, qy, qz = get_q(last)
        p2_ref[:, pl.ds(i - 1, 1), 0:1] = qx
        p2_ref[:, pl.ds(i - 1, 1), 1:2] = qy
        p2_ref[:, pl.ds(i - 1, 1), 2:3] = qz
        d = (px - qx) ** 2 + (py - qy) ** 2 + (pz - qz) ** 2
        dists = jnp.minimum(dists, d)
        mx = jnp.max(dists, axis=(1, 2), keepdims=True)
        nxt = jnp.min(jnp.where(dists == mx, flat, N),
                      axis=(1, 2), keepdims=True)
        return dists, nxt

    dists0 = jnp.full((B, R, L), 1e10, dtype=jnp.float32)
    last0 = jnp.zeros((B, 1, 1), dtype=jnp.int32)
    _, last = lax.fori_loop(1, M, body, (dists0, last0), unroll=2)
    qx, qy, qz = get_q(last)
    p2_ref[:, M - 1:M, 0:1] = qx
    p2_ref[:, M - 1:M, 1:2] = qy
    p2_ref[:, M - 1:M, 2:3] = qz


def _knn_body(pp8_ref, q8_ref, out_ref):
    b = pl.program_id(0)
    q = q8_ref[0]
    P = pp8_ref[0]
    qn = jnp.sum(q * q, axis=1, keepdims=True)
    pn = jnp.sum(P * P, axis=0, keepdims=True)
    dot = jnp.dot(q, P, preferred_element_type=jnp.float32)
    D = qn + pn - 2.0 * dot
    H = N // 4
    iota2 = lax.broadcasted_iota(jnp.int32, (1, H), 1)
    ch = [D[:, i * H:(i + 1) * H] for i in range(4)]
    rank = []
    for jj in range(4):
        r = jnp.zeros((MB, H), jnp.int32)
        for ii in range(4):
            if ii == jj:
                continue
            if ii < jj:
                r = r + (ch[ii] <= ch[jj]).astype(jnp.int32)
            else:
                r = r + (ch[ii] < ch[jj]).astype(jnp.int32)
        rank.append(r)
    vs, ns = [], []
    for rr, kk in enumerate((K, 8, 4, 1)):
        sv = jnp.full((MB, H), jnp.inf, jnp.float32)
        fv = jnp.zeros((MB, H), jnp.int32)
        for jj in range(4):
            m = rank[jj] == rr
            sv = jnp.where(m, ch[jj], sv)
            fv = jnp.where(m, iota2 + jj * H, fv)
        for j in range(kk):
            mn = jnp.min(sv, axis=1, keepdims=True)
            nx = jnp.min(jnp.where(sv == mn, fv, N), axis=1, keepdims=True)
            vs.append(mn)
            ns.append(nx)
            sv = jnp.where(fv == nx, jnp.inf, sv)
    Vc = jnp.concatenate(vs, axis=1)
    Nc = jnp.concatenate(ns, axis=1)
    for j in range(K):
        mn = jnp.min(Vc, axis=1, keepdims=True)
        nx = jnp.min(jnp.where(Vc == mn, Nc, N), axis=1, keepdims=True)
        out_ref[0, :, j:j + 1] = nx + b * N
        Vc = jnp.where((Vc == mn) & (Nc == nx), jnp.inf, Vc)


def _make_sc_gather(rows, dp):
    info = plsc.get_sparse_core_info()
    nw = info.num_cores * info.num_subcores
    per_w = rows // nw
    chunk = 128
    mesh = plsc.VectorSubcoreMesh(core_axis_name="c", subcore_axis_name="s")

    @functools.partial(
        pl.kernel,
        mesh=mesh,
        out_type=jax.ShapeDtypeStruct((rows, dp), jnp.float32),
        scratch_types=[
            pltpu.VMEM((chunk,), jnp.int32),
            pltpu.VMEM((chunk, dp), jnp.float32),
            pltpu.SemaphoreType.DMA,
        ],
    )
    def gather_k(table_hbm, idx_hbm, out_hbm, idx_v, rows_v, sem):
        wid = lax.axis_index("s") * info.num_cores + lax.axis_index("c")
        base = wid * per_w

        def body(ci, carry):
            off = base + ci * chunk
            pltpu.sync_copy(idx_hbm.at[pl.ds(off, chunk)], idx_v)
            pltpu.async_copy(table_hbm.at[idx_v], rows_v, sem).wait()
            pltpu.sync_copy(rows_v, out_hbm.at[pl.ds(off, chunk)])
            return carry

        lax.fori_loop(0, per_w // chunk, body, 0)

    return gather_k


def _mlp1_body(g_ref, q8_ref, w1p_ref, w1x_ref, h1_ref, st_ref):
    i = pl.program_id(0)
    g = g_ref[...]
    h = jnp.dot(g.reshape(MB2 * K, DP), w1p_ref[...],
                preferred_element_type=jnp.float32)
    corr = jnp.dot(q8_ref[...], w1x_ref[...],
                   preferred_element_type=jnp.float32)
    h = h.reshape(MB2, K, OUT) - corr[:, None, :]
    h1_ref[...] = h

    @pl.when(i == 0)
    def _():
        st_ref[...] = jnp.zeros((8, 128), jnp.float32)

    st_ref[0:1, :] = st_ref[0:1, :] + jnp.sum(h, axis=(0, 1))[None]
    st_ref[1:2, :] = st_ref[1:2, :] + jnp.sum(h * h, axis=(0, 1))[None]


def _mlp2_body(h1_ref, st1_ref, aux_ref, w2t_ref, hmx_ref, hmn_ref, st_ref):
    i = pl.program_id(0)
    inv = 1.0 / S
    mean1 = st1_ref[0:1, :] * inv
    var1 = st1_ref[1:2, :] * inv - mean1 * mean1
    s1 = aux_ref[0:1, :] * lax.rsqrt(var1 + EPS)
    t1 = aux_ref[1:2, :] - mean1 * s1
    h1 = h1_ref[...]
    a = jnp.maximum(h1 * s1[0] + t1[0], 0.0)
    h2 = jnp.dot(a.reshape(MB2 * K, OUT), w2t_ref[...],
                 preferred_element_type=jnp.float32).reshape(MB2, K, OUT)
    hmx_ref[...] = jnp.max(h2, axis=1)
    hmn_ref[...] = jnp.min(h2, axis=1)

    @pl.when(i == 0)
    def _():
        st_ref[...] = jnp.zeros((8, 128), jnp.float32)

    st_ref[0:1, :] = st_ref[0:1, :] + jnp.sum(h2, axis=(0, 1))[None]
    st_ref[1:2, :] = st_ref[1:2, :] + jnp.sum(h2 * h2, axis=(0, 1))[None]


def _final_body(hmx_ref, hmn_ref, st2_ref, aux_ref, y_ref):
    inv = 1.0 / S
    mean2 = st2_ref[0:1, :] * inv
    var2 = st2_ref[1:2, :] * inv - mean2 * mean2
    s2 = aux_ref[2:3, :] * lax.rsqrt(var2 + EPS)
    t2 = aux_ref[3:4, :] - mean2 * s2
    z = jnp.maximum(hmx_ref[...] * s2[0], hmn_ref[...] * s2[0])
    y_ref[...] = jnp.maximum(z + t2[0], 0.0)


def kernel(p1, x, W1, g1, b1, W2, g2, b2):
    f32 = jnp.float32
    pt = jnp.transpose(p1, (0, 2, 1))
    pp = pt.reshape(B, 3, R, L)
    pp8 = jnp.concatenate([pt, jnp.zeros((B, 5, N), f32)], axis=1)

    p2 = pl.pallas_call(
        _fps_body,
        out_shape=jax.ShapeDtypeStruct((B, M, 3), f32),
    )(pp)

    p2p8 = jnp.concatenate([p2, jnp.zeros((B, M, 5), f32)], axis=2)
    knn_flat = pl.pallas_call(
        _knn_body,
        grid=(B, M // MB),
        in_specs=[
            pl.BlockSpec((1, 8, N), lambda b, j: (b, 0, 0)),
            pl.BlockSpec((1, MB, 8), lambda b, j: (b, j, 0)),
        ],
        out_specs=pl.BlockSpec((1, MB, K), lambda b, j: (b, j, 0)),
        out_shape=jax.ShapeDtypeStruct((B, M, K), jnp.int32),
    )(pp8, p2p8)

    table = jnp.concatenate(
        [p1, jnp.transpose(x, (0, 2, 1)), jnp.zeros((B, N, DP - 3 - C), f32)],
        axis=2).reshape(B * N, DP)
    idx_flat = knn_flat.reshape(B * M * K)
    g = _make_sc_gather(B * M * K, DP)(table, idx_flat)
    g3 = g.reshape(B * M, K, DP)

    w1p = jnp.pad(W1, ((0, 0), (0, DP - 3 - C))).T
    w1x = jnp.pad(W1[:, :3], ((0, 0), (0, 5))).T
    q8f = p2p8.reshape(B * M, 8)
    h1, st1 = pl.pallas_call(
        _mlp1_body,
        grid=(G2,),
        in_specs=[
            pl.BlockSpec((MB2, K, DP), lambda i: (i, 0, 0)),
            pl.BlockSpec((MB2, 8), lambda i: (i, 0)),
            pl.BlockSpec((DP, OUT), lambda i: (0, 0)),
            pl.BlockSpec((8, OUT), lambda i: (0, 0)),
        ],
        out_specs=[
            pl.BlockSpec((MB2, K, OUT), lambda i: (i, 0, 0)),
            pl.BlockSpec((8, 128), lambda i: (0, 0)),
        ],
        out_shape=[
            jax.ShapeDtypeStruct((B * M, K, OUT), f32),
            jax.ShapeDtypeStruct((8, 128), f32),
        ],
    )(g3, q8f, w1p, w1x)

    aux = jnp.concatenate(
        [g1[None], b1[None], g2[None], b2[None],
         jnp.zeros((4, OUT), f32)], axis=0)
    hmx, hmn, st2 = pl.pallas_call(
        _mlp2_body,
        grid=(G2,),
        in_specs=[
            pl.BlockSpec((MB2, K, OUT), lambda i: (i, 0, 0)),
            pl.BlockSpec((8, 128), lambda i: (0, 0)),
            pl.BlockSpec((8, 128), lambda i: (0, 0)),
            pl.BlockSpec((OUT, OUT), lambda i: (0, 0)),
        ],
        out_specs=[
            pl.BlockSpec((MB2, OUT), lambda i: (i, 0)),
            pl.BlockSpec((MB2, OUT), lambda i: (i, 0)),
            pl.BlockSpec((8, 128), lambda i: (0, 0)),
        ],
        out_shape=[
            jax.ShapeDtypeStruct((B * M, OUT), f32),
            jax.ShapeDtypeStruct((B * M, OUT), f32),
            jax.ShapeDtypeStruct((8, 128), f32),
        ],
    )(h1, st1, aux, W2.T)

    y2 = pl.pallas_call(
        _final_body,
        grid=(G2,),
        in_specs=[
            pl.BlockSpec((MB2, OUT), lambda i: (i, 0)),
            pl.BlockSpec((MB2, OUT), lambda i: (i, 0)),
            pl.BlockSpec((8, 128), lambda i: (0, 0)),
            pl.BlockSpec((8, 128), lambda i: (0, 0)),
        ],
        out_specs=pl.BlockSpec((MB2, OUT), lambda i: (i, 0)),
        out_shape=jax.ShapeDtypeStruct((B * M, OUT), f32),
    )(hmx, hmn, st2, aux)

    y = jnp.transpose(y2.reshape(B, M, OUT), (0, 2, 1))
    return (p2, y)
